# baseline (speedup 1.0000x reference)
import functools

import jax
import jax.numpy as jnp
from jax import lax
from jax.experimental import pallas as pl
from jax.experimental.pallas import tpu as pltpu
from jax.experimental.pallas import tpu_sc as plsc

D = 64
NC, NS = 2, 16
NW = NC * NS
CHUNK = 128
NBUF = 4

PB = 512
CB = 2 * PB


def _repack_body(t_ref, o_ref):
    blk = t_ref[...]
    a = blk[:, :PB].T
    b = blk[:, PB:].T
    o_ref[...] = jnp.concatenate([a, b], axis=1)


def _repack(table_t, vocab):
    nblk = pl.cdiv(vocab, CB)
    out2 = pl.pallas_call(
        _repack_body,
        grid=(nblk,),
        in_specs=[pl.BlockSpec((D, CB), lambda i: (0, i))],
        out_specs=pl.BlockSpec((PB, 128), lambda i: (i, 0)),
        out_shape=jax.ShapeDtypeStruct((nblk * PB, 128), jnp.float32),
    )(table_t)
    return out2.reshape(nblk * PB * 2, D)


@functools.lru_cache(maxsize=None)
def _make_gather(B, vpad):
    assert B % (NW * CHUNK * NBUF) == 0
    b_per_w = B // NW
    n_chunks = b_per_w // CHUNK
    mesh = plsc.VectorSubcoreMesh(core_axis_name="c", subcore_axis_name="s")

    @functools.partial(
        pl.kernel,
        mesh=mesh,
        out_type=jax.ShapeDtypeStruct((B, D), jnp.float32),
        compiler_params=pltpu.CompilerParams(use_tc_tiling_on_sc=False),
        scratch_types=[
            pltpu.VMEM((b_per_w,), jnp.int32),
            pltpu.VMEM((NBUF, CHUNK, D), jnp.float32),
            pltpu.SemaphoreType.DMA((NBUF,)),
        ],
    )
    def k(idx_hbm, table_hbm, out_hbm, idx_v, bufs, sems):
        wid = lax.axis_index("s") * NC + lax.axis_index("c")
        base = wid * b_per_w
        pltpu.sync_copy(idx_hbm.at[pl.ds(base, b_per_w)], idx_v)

        def remap(j, carry):
            v = idx_v[pl.ds(j * 16, 16)]
            lin = (v & ~1023) | ((v & 511) << 1) | ((v >> 9) & 1)
            idx_v[pl.ds(j * 16, 16)] = lin
            return carry

        lax.fori_loop(0, b_per_w // 16, remap, 0)

        def gather(i, b):
            pltpu.make_async_copy(
                table_hbm.at[idx_v.at[pl.ds(i * CHUNK, CHUNK)]],
                bufs.at[b],
                sems.at[b],
            ).start()

        for b in range(NBUF):
            gather(b, b)

        def body(g, carry):
            c = g * NBUF
            for b in range(NBUF):
                i = c + b
                pltpu.make_async_copy(
                    table_hbm.at[idx_v.at[pl.ds(0, CHUNK)]],
                    bufs.at[b],
                    sems.at[b],
                ).wait()
                pltpu.sync_copy(
                    bufs.at[b], out_hbm.at[pl.ds(base + i * CHUNK, CHUNK)]
                )
                nxt = i + NBUF

                @pl.when(nxt < n_chunks)
                def _():
                    gather(nxt, b)

            return carry

        lax.fori_loop(0, n_chunks // NBUF, body, 0)

    return k


@jax.jit
def kernel(x, table):
    r, c = x.shape
    B = r * c
    vocab = table.shape[0]
    x_flat = x.reshape(B).astype(jnp.int32)
    table_lin = _repack(table.T, vocab)
    out = _make_gather(B, table_lin.shape[0])(x_flat, table_lin)
    return out.reshape(r, c, D)


# speedup vs baseline: 1.5790x; 1.5790x over previous
import functools

import jax
import jax.numpy as jnp
from jax import lax
from jax.experimental import pallas as pl
from jax.experimental.pallas import tpu as pltpu
from jax.experimental.pallas import tpu_sc as plsc

D = 64
NC, NS = 2, 16
NW = NC * NS
CHUNK = 128
NBUF = 4
SHIFT = 11

PB = 2048
CB = 2 * PB


def _repack_body(t_ref, o_ref):
    blk = t_ref[...]
    a = blk[:, :PB].T
    b = blk[:, PB:].T
    o_ref[...] = jnp.concatenate([a, b], axis=1)


def _repack(table_t, vocab):
    nblk = pl.cdiv(vocab, CB)
    out2 = pl.pallas_call(
        _repack_body,
        grid=(nblk,),
        in_specs=[pl.BlockSpec((D, CB), lambda i: (0, i))],
        out_specs=pl.BlockSpec((PB, 128), lambda i: (i, 0)),
        out_shape=jax.ShapeDtypeStruct((nblk * PB, 128), jnp.float32),
    )(table_t)
    return out2.reshape(nblk * PB * 2, D)


@functools.lru_cache(maxsize=None)
def _make_gather(B, vpad):
    assert B % (NW * CHUNK * NBUF) == 0
    b_per_w = B // NW
    n_chunks = b_per_w // CHUNK
    mesh = plsc.VectorSubcoreMesh(core_axis_name="c", subcore_axis_name="s")

    @functools.partial(
        pl.kernel,
        mesh=mesh,
        out_type=jax.ShapeDtypeStruct((B, D), jnp.float32),
        compiler_params=pltpu.CompilerParams(use_tc_tiling_on_sc=False),
        scratch_types=[
            pltpu.VMEM((b_per_w,), jnp.int32),
            pltpu.VMEM((NBUF, CHUNK, D), jnp.float32),
            pltpu.SemaphoreType.DMA((NBUF,)),
        ],
    )
    def k(idx_hbm, table_hbm, out_hbm, idx_v, bufs, sems):
        wid = lax.axis_index("s") * NC + lax.axis_index("c")
        base = wid * b_per_w
        pltpu.sync_copy(idx_hbm.at[pl.ds(base, b_per_w)], idx_v)

        def remap(j, carry):
            v = idx_v[pl.ds(j * 16, 16)]
            lin = (v & ~(2 * PB - 1)) | ((v & (PB - 1)) << 1) | ((v >> SHIFT) & 1)
            idx_v[pl.ds(j * 16, 16)] = lin
            return carry

        lax.fori_loop(0, b_per_w // 16, remap, 0)

        def gather(i, b):
            pltpu.make_async_copy(
                table_hbm.at[idx_v.at[pl.ds(i * CHUNK, CHUNK)]],
                bufs.at[b],
                sems.at[b],
            ).start()

        for b in range(NBUF):
            gather(b, b)

        def body(g, carry):
            c = g * NBUF
            for b in range(NBUF):
                i = c + b
                pltpu.make_async_copy(
                    table_hbm.at[idx_v.at[pl.ds(0, CHUNK)]],
                    bufs.at[b],
                    sems.at[b],
                ).wait()
                pltpu.sync_copy(
                    bufs.at[b], out_hbm.at[pl.ds(base + i * CHUNK, CHUNK)]
                )
                nxt = i + NBUF

                @pl.when(nxt < n_chunks)
                def _():
                    gather(nxt, b)

            return carry

        lax.fori_loop(0, n_chunks // NBUF, body, 0)

    return k


@jax.jit
def kernel(x, table):
    r, c = x.shape
    B = r * c
    vocab = table.shape[0]
    x_flat = x.reshape(B).astype(jnp.int32)
    table_lin = _repack(table.T, vocab)
    out = _make_gather(B, table_lin.shape[0])(x_flat, table_lin)
    return out.reshape(r, c, D)


# speedup vs baseline: 1.8707x; 1.1847x over previous
import functools

import jax
import jax.numpy as jnp
from jax import lax
from jax.experimental import pallas as pl
from jax.experimental.pallas import tpu as pltpu
from jax.experimental.pallas import tpu_sc as plsc

D = 64
NC, NS = 2, 16
NW = NC * NS
CHUNK = 128
NBUF = 4
SHIFT = 13

PB = 8192
CB = 2 * PB


def _repack_body(t_ref, o_ref):
    blk = t_ref[...]
    a = blk[:, :PB].T
    b = blk[:, PB:].T
    o_ref[...] = jnp.concatenate([a, b], axis=1)


def _repack(table_t, vocab):
    nblk = pl.cdiv(vocab, CB)
    out2 = pl.pallas_call(
        _repack_body,
        grid=(nblk,),
        in_specs=[pl.BlockSpec((D, CB), lambda i: (0, i))],
        out_specs=pl.BlockSpec((PB, 128), lambda i: (i, 0)),
        out_shape=jax.ShapeDtypeStruct((nblk * PB, 128), jnp.float32),
    )(table_t)
    return out2.reshape(nblk * PB * 2, D)


@functools.lru_cache(maxsize=None)
def _make_gather(B, vpad):
    assert B % (NW * CHUNK * NBUF) == 0
    b_per_w = B // NW
    n_chunks = b_per_w // CHUNK
    mesh = plsc.VectorSubcoreMesh(core_axis_name="c", subcore_axis_name="s")

    @functools.partial(
        pl.kernel,
        mesh=mesh,
        out_type=jax.ShapeDtypeStruct((B, D), jnp.float32),
        compiler_params=pltpu.CompilerParams(use_tc_tiling_on_sc=False),
        scratch_types=[
            pltpu.VMEM((b_per_w,), jnp.int32),
            pltpu.VMEM((NBUF, CHUNK, D), jnp.float32),
            pltpu.SemaphoreType.DMA((NBUF,)),
        ],
    )
    def k(idx_hbm, table_hbm, out_hbm, idx_v, bufs, sems):
        wid = lax.axis_index("s") * NC + lax.axis_index("c")
        base = wid * b_per_w
        pltpu.sync_copy(idx_hbm.at[pl.ds(base, b_per_w)], idx_v)

        def remap(j, carry):
            v = idx_v[pl.ds(j * 16, 16)]
            lin = (v & ~(2 * PB - 1)) | ((v & (PB - 1)) << 1) | ((v >> SHIFT) & 1)
            idx_v[pl.ds(j * 16, 16)] = lin
            return carry

        lax.fori_loop(0, b_per_w // 16, remap, 0)

        def gather(i, b):
            pltpu.make_async_copy(
                table_hbm.at[idx_v.at[pl.ds(i * CHUNK, CHUNK)]],
                bufs.at[b],
                sems.at[b],
            ).start()

        for b in range(NBUF):
            gather(b, b)

        def body(g, carry):
            c = g * NBUF
            for b in range(NBUF):
                i = c + b
                pltpu.make_async_copy(
                    table_hbm.at[idx_v.at[pl.ds(0, CHUNK)]],
                    bufs.at[b],
                    sems.at[b],
                ).wait()
                pltpu.sync_copy(
                    bufs.at[b], out_hbm.at[pl.ds(base + i * CHUNK, CHUNK)]
                )
                nxt = i + NBUF

                @pl.when(nxt < n_chunks)
                def _():
                    gather(nxt, b)

            return carry

        lax.fori_loop(0, n_chunks // NBUF, body, 0)

    return k


@jax.jit
def kernel(x, table):
    r, c = x.shape
    B = r * c
    vocab = table.shape[0]
    x_flat = x.reshape(B).astype(jnp.int32)
    table_lin = _repack(table.T, vocab)
    out = _make_gather(B, table_lin.shape[0])(x_flat, table_lin)
    return out.reshape(r, c, D)


# speedup vs baseline: 1.9236x; 1.0283x over previous
import functools

import jax
import jax.numpy as jnp
from jax import lax
from jax.experimental import pallas as pl
from jax.experimental.pallas import tpu as pltpu
from jax.experimental.pallas import tpu_sc as plsc

D = 64
NC, NS = 2, 16
NW = NC * NS
CHUNK = 128
NBUF = 4
SHIFT = 14

PB = 16384
CB = 2 * PB


def _repack_body(t_ref, o_ref):
    blk = t_ref[...]
    a = blk[:, :PB].T
    b = blk[:, PB:].T
    o_ref[...] = jnp.concatenate([a, b], axis=1)


def _repack(table_t, vocab):
    nblk = pl.cdiv(vocab, CB)
    out2 = pl.pallas_call(
        _repack_body,
        grid=(nblk,),
        in_specs=[pl.BlockSpec((D, CB), lambda i: (0, i))],
        out_specs=pl.BlockSpec((PB, 128), lambda i: (i, 0)),
        out_shape=jax.ShapeDtypeStruct((nblk * PB, 128), jnp.float32),
    )(table_t)
    return out2.reshape(nblk * PB * 2, D)


@functools.lru_cache(maxsize=None)
def _make_gather(B, vpad):
    assert B % (NW * CHUNK * NBUF) == 0
    b_per_w = B // NW
    n_chunks = b_per_w // CHUNK
    mesh = plsc.VectorSubcoreMesh(core_axis_name="c", subcore_axis_name="s")

    @functools.partial(
        pl.kernel,
        mesh=mesh,
        out_type=jax.ShapeDtypeStruct((B, D), jnp.float32),
        compiler_params=pltpu.CompilerParams(use_tc_tiling_on_sc=False),
        scratch_types=[
            pltpu.VMEM((b_per_w,), jnp.int32),
            pltpu.VMEM((NBUF, CHUNK, D), jnp.float32),
            pltpu.SemaphoreType.DMA((NBUF,)),
        ],
    )
    def k(idx_hbm, table_hbm, out_hbm, idx_v, bufs, sems):
        wid = lax.axis_index("s") * NC + lax.axis_index("c")
        base = wid * b_per_w
        pltpu.sync_copy(idx_hbm.at[pl.ds(base, b_per_w)], idx_v)

        def remap(j, carry):
            v = idx_v[pl.ds(j * 16, 16)]
            lin = (v & ~(2 * PB - 1)) | ((v & (PB - 1)) << 1) | ((v >> SHIFT) & 1)
            idx_v[pl.ds(j * 16, 16)] = lin
            return carry

        lax.fori_loop(0, b_per_w // 16, remap, 0)

        def gather(i, b):
            pltpu.make_async_copy(
                table_hbm.at[idx_v.at[pl.ds(i * CHUNK, CHUNK)]],
                bufs.at[b],
                sems.at[b],
            ).start()

        for b in range(NBUF):
            gather(b, b)

        def body(g, carry):
            c = g * NBUF
            for b in range(NBUF):
                i = c + b
                pltpu.make_async_copy(
                    table_hbm.at[idx_v.at[pl.ds(0, CHUNK)]],
                    bufs.at[b],
                    sems.at[b],
                ).wait()
                pltpu.sync_copy(
                    bufs.at[b], out_hbm.at[pl.ds(base + i * CHUNK, CHUNK)]
                )
                nxt = i + NBUF

                @pl.when(nxt < n_chunks)
                def _():
                    gather(nxt, b)

            return carry

        lax.fori_loop(0, n_chunks // NBUF, body, 0)

    return k


@jax.jit
def kernel(x, table):
    r, c = x.shape
    B = r * c
    vocab = table.shape[0]
    x_flat = x.reshape(B).astype(jnp.int32)
    table_lin = _repack(table.T, vocab)
    out = _make_gather(B, table_lin.shape[0])(x_flat, table_lin)
    return out.reshape(r, c, D)


# speedup vs baseline: 1.9239x; 1.0001x over previous
import functools

import jax
import jax.numpy as jnp
from jax import lax
from jax.experimental import pallas as pl
from jax.experimental.pallas import tpu as pltpu
from jax.experimental.pallas import tpu_sc as plsc

D = 64
NC, NS = 2, 16
NW = NC * NS
CHUNK = 256
NBUF = 4
SHIFT = 14

PB = 16384
CB = 2 * PB


def _repack_body(t_ref, o_ref):
    blk = t_ref[...]
    a = blk[:, :PB].T
    b = blk[:, PB:].T
    o_ref[...] = jnp.concatenate([a, b], axis=1)


def _repack(table_t, vocab):
    nblk = pl.cdiv(vocab, CB)
    out2 = pl.pallas_call(
        _repack_body,
        grid=(nblk,),
        in_specs=[pl.BlockSpec((D, CB), lambda i: (0, i))],
        out_specs=pl.BlockSpec((PB, 128), lambda i: (i, 0)),
        out_shape=jax.ShapeDtypeStruct((nblk * PB, 128), jnp.float32),
    )(table_t)
    return out2.reshape(nblk * PB * 2, D)


@functools.lru_cache(maxsize=None)
def _make_gather(B, vpad):
    assert B % (NW * CHUNK * NBUF) == 0
    b_per_w = B // NW
    n_chunks = b_per_w // CHUNK
    mesh = plsc.VectorSubcoreMesh(core_axis_name="c", subcore_axis_name="s")

    @functools.partial(
        pl.kernel,
        mesh=mesh,
        out_type=jax.ShapeDtypeStruct((B, D), jnp.float32),
        compiler_params=pltpu.CompilerParams(use_tc_tiling_on_sc=False),
        scratch_types=[
            pltpu.VMEM((b_per_w,), jnp.int32),
            pltpu.VMEM((NBUF, CHUNK, D), jnp.float32),
            pltpu.SemaphoreType.DMA((NBUF,)),
        ],
    )
    def k(idx_hbm, table_hbm, out_hbm, idx_v, bufs, sems):
        wid = lax.axis_index("s") * NC + lax.axis_index("c")
        base = wid * b_per_w
        pltpu.sync_copy(idx_hbm.at[pl.ds(base, b_per_w)], idx_v)

        def remap(j, carry):
            v = idx_v[pl.ds(j * 16, 16)]
            lin = (v & ~(2 * PB - 1)) | ((v & (PB - 1)) << 1) | ((v >> SHIFT) & 1)
            idx_v[pl.ds(j * 16, 16)] = lin
            return carry

        lax.fori_loop(0, b_per_w // 16, remap, 0)

        def gather(i, b):
            pltpu.make_async_copy(
                table_hbm.at[idx_v.at[pl.ds(i * CHUNK, CHUNK)]],
                bufs.at[b],
                sems.at[b],
            ).start()

        for b in range(NBUF):
            gather(b, b)

        def body(g, carry):
            c = g * NBUF
            for b in range(NBUF):
                i = c + b
                pltpu.make_async_copy(
                    table_hbm.at[idx_v.at[pl.ds(0, CHUNK)]],
                    bufs.at[b],
                    sems.at[b],
                ).wait()
                pltpu.sync_copy(
                    bufs.at[b], out_hbm.at[pl.ds(base + i * CHUNK, CHUNK)]
                )
                nxt = i + NBUF

                @pl.when(nxt < n_chunks)
                def _():
                    gather(nxt, b)

            return carry

        lax.fori_loop(0, n_chunks // NBUF, body, 0)

    return k


@jax.jit
def kernel(x, table):
    r, c = x.shape
    B = r * c
    vocab = table.shape[0]
    x_flat = x.reshape(B).astype(jnp.int32)
    table_lin = _repack(table.T, vocab)
    out = _make_gather(B, table_lin.shape[0])(x_flat, table_lin)
    return out.reshape(r, c, D)
